# Initial kernel scaffold; baseline (speedup 1.0000x reference)
import functools

import jax
import jax.numpy as jnp
from jax import lax
from jax.experimental import pallas as pl
from jax.experimental.pallas import tpu as pltpu
from jax.experimental.pallas import tpu_sc as plsc

N_NODES = 10000
D = 128
E_EDGES = 320000
NC, NS = 2, 16
NW = NC * NS
EC = 2 * E_EDGES
E_PER = EC // NW
CHUNK = 80
NCHUNK = E_PER // CHUNK
ROWS_PER_TILE = N_NODES // NS
ZROWS = 125
LANES = 16


def _sc_body(h_hbm, row_hbm, col_hbm, w_hbm, out_hbm,
             row_v, col_v, w_v, gbuf, zbuf, acc, sem):
    c = lax.axis_index("c")
    s = lax.axis_index("s")
    wid = c * NS + s

    pltpu.sync_copy(row_hbm.at[wid], row_v)
    pltpu.sync_copy(col_hbm.at[wid], col_v)
    pltpu.sync_copy(w_hbm.at[wid], w_v)

    zero = jnp.zeros((LANES,), jnp.float32)

    def zrow(i, carry):
        for j in range(D // LANES):
            zbuf[i, pl.ds(j * LANES, LANES)] = zero
        return carry

    lax.fori_loop(0, ZROWS, zrow, 0)
    for z in range(ROWS_PER_TILE // ZROWS):
        pltpu.sync_copy(
            zbuf, acc.at[pl.ds(s * ROWS_PER_TILE + z * ZROWS, ZROWS)])
    plsc.subcore_barrier()

    def chunk_body(k, carry):
        pltpu.async_copy(h_hbm.at[col_v.at[k]], gbuf, sem).wait()

        def edge_body(r, inner):
            wv = plsc.load_gather(
                w_v, [jnp.full((LANES,), k, jnp.int32),
                      jnp.full((LANES,), r, jnp.int32)])
            for j in range(D // LANES):
                gbuf[r, pl.ds(j * LANES, LANES)] = (
                    gbuf[r, pl.ds(j * LANES, LANES)] * wv)
            return inner

        lax.fori_loop(0, CHUNK, edge_body, 0)

        pltpu.sync_copy(gbuf, acc.at[row_v.at[k]], add=True)
        return carry

    lax.fori_loop(0, NCHUNK, chunk_body, 0)

    plsc.subcore_barrier()
    pltpu.sync_copy(acc.at[pl.ds(s * ROWS_PER_TILE, ROWS_PER_TILE)],
                    out_hbm.at[c, pl.ds(s * ROWS_PER_TILE, ROWS_PER_TILE)])


_sc_aggregate = functools.partial(
    pl.kernel,
    out_type=jax.ShapeDtypeStruct((NC, N_NODES, D), jnp.float32),
    mesh=plsc.VectorSubcoreMesh(core_axis_name="c", subcore_axis_name="s"),
    scratch_types=[
        pltpu.VMEM((NCHUNK, CHUNK), jnp.int32),
        pltpu.VMEM((NCHUNK, CHUNK), jnp.int32),
        pltpu.VMEM((NCHUNK, CHUNK), jnp.float32),
        pltpu.VMEM((CHUNK, D), jnp.float32),
        pltpu.VMEM((ZROWS, D), jnp.float32),
        pltpu.VMEM_SHARED((N_NODES, D), jnp.float32),
        pltpu.SemaphoreType.DMA,
    ],
)(_sc_body)


BM = 2500


def _mm_body(x_ref, w1_ref, b1_ref, w2_ref, b2_ref, o_ref):
    xb = x_ref[...]
    dims = (((1,), (1,)), ((), ()))
    o_ref[0] = lax.dot_general(
        xb, w1_ref[...], dims, preferred_element_type=jnp.float32) + b1_ref[...]
    o_ref[1] = lax.dot_general(
        xb, w2_ref[...], dims, preferred_element_type=jnp.float32) + b2_ref[...]


def _hidden(x, W1, b1, W2, b2):
    h = pl.pallas_call(
        _mm_body,
        grid=(N_NODES // BM,),
        in_specs=[
            pl.BlockSpec((BM, D), lambda i: (i, 0)),
            pl.BlockSpec((D, D), lambda i: (0, 0)),
            pl.BlockSpec((1, D), lambda i: (0, 0)),
            pl.BlockSpec((D, D), lambda i: (0, 0)),
            pl.BlockSpec((1, D), lambda i: (0, 0)),
        ],
        out_specs=pl.BlockSpec((2, BM, D), lambda i: (0, i, 0)),
        out_shape=jax.ShapeDtypeStruct((2, N_NODES, D), jnp.float32),
    )(x, W1, b1.reshape(1, D), W2, b2.reshape(1, D))
    return h.reshape(2 * N_NODES, D)


def _add_body(p_ref, o_ref):
    o_ref[...] = p_ref[0] + p_ref[1]


def _final_add(partials):
    return pl.pallas_call(
        _add_body,
        grid=(N_NODES // BM,),
        in_specs=[pl.BlockSpec((2, BM, D), lambda i: (0, i, 0))],
        out_specs=pl.BlockSpec((BM, D), lambda i: (i, 0)),
        out_shape=jax.ShapeDtypeStruct((N_NODES, D), jnp.float32),
    )(partials)


def kernel(x, edge_index1, edge_weight1, edge_index2, edge_weight2,
           W1, b1, W2, b2):
    rows = jnp.concatenate(
        [edge_index1[0], edge_index2[0]]).reshape(NW, NCHUNK, CHUNK)
    cols = jnp.concatenate(
        [edge_index1[1], edge_index2[1] + N_NODES]).reshape(NW, NCHUNK, CHUNK)
    w = jnp.concatenate(
        [edge_weight1, edge_weight2]).reshape(NW, NCHUNK, CHUNK)
    H = _hidden(x, W1, b1, W2, b2)
    partials = _sc_aggregate(H, rows, cols, w)
    return _final_add(partials)


# speedup vs baseline: 2.1468x; 2.1468x over previous
import functools

import jax
import jax.numpy as jnp
from jax import lax
from jax.experimental import pallas as pl
from jax.experimental.pallas import tpu as pltpu
from jax.experimental.pallas import tpu_sc as plsc

N_NODES = 10000
D = 128
E_EDGES = 320000
NC, NS = 2, 16
NW = NC * NS
CHUNK = 128
NCHUNK = 160
SEG = 32
NSEG = NCHUNK // SEG
E_PER = NCHUNK * CHUNK
EC_PAD = NW * E_PER
N_PAD = 10240
ROWS_PER_TILE = N_PAD // NS
LANES = 16


def _sc_body(h_hbm, row_hbm, col_hbm, w_hbm, out_hbm,
             row_v, col_v, w_v, gbuf, acc, sem):
    c = lax.axis_index("c")
    s = lax.axis_index("s")
    wid = c * NS + s

    zero = jnp.zeros((LANES,), jnp.float32)

    def zrow(i, carry):
        for j in range(D // LANES):
            gbuf[i, pl.ds(j * LANES, LANES)] = zero
        return carry

    lax.fori_loop(0, CHUNK, zrow, 0)
    for z in range(ROWS_PER_TILE // CHUNK):
        pltpu.sync_copy(
            gbuf, acc.at[pl.ds(s * ROWS_PER_TILE + z * CHUNK, CHUNK)])
    plsc.subcore_barrier()

    def seg_body(g, carry):
        pltpu.sync_copy(row_hbm.at[wid, pl.ds(g * SEG, SEG)], row_v)
        pltpu.sync_copy(col_hbm.at[wid, pl.ds(g * SEG, SEG)], col_v)
        pltpu.sync_copy(w_hbm.at[wid, pl.ds(g * SEG, SEG)], w_v)

        def chunk_body(k, inner):
            pltpu.async_copy(h_hbm.at[col_v.at[k]], gbuf, sem).wait()

            for b in range(CHUNK // LANES):
                wv16 = w_v[k, pl.ds(b * LANES, LANES)]
                for l in range(LANES):
                    wv = jnp.full((LANES,), wv16[l], jnp.float32)
                    r = b * LANES + l
                    for j in range(D // LANES):
                        gbuf[r, pl.ds(j * LANES, LANES)] = (
                            gbuf[r, pl.ds(j * LANES, LANES)] * wv)

            pltpu.sync_copy(gbuf, acc.at[row_v.at[k]], add=True)
            return inner

        lax.fori_loop(0, SEG, chunk_body, 0)
        return carry

    lax.fori_loop(0, NSEG, seg_body, 0)

    plsc.subcore_barrier()
    pltpu.sync_copy(acc.at[pl.ds(s * ROWS_PER_TILE, ROWS_PER_TILE)],
                    out_hbm.at[c, pl.ds(s * ROWS_PER_TILE, ROWS_PER_TILE)])


_sc_aggregate = functools.partial(
    pl.kernel,
    out_type=jax.ShapeDtypeStruct((NC, N_PAD, D), jnp.float32),
    mesh=plsc.VectorSubcoreMesh(core_axis_name="c", subcore_axis_name="s"),
    scratch_types=[
        pltpu.VMEM((SEG, CHUNK), jnp.int32),
        pltpu.VMEM((SEG, CHUNK), jnp.int32),
        pltpu.VMEM((SEG, CHUNK), jnp.float32),
        pltpu.VMEM((CHUNK, D), jnp.float32),
        pltpu.VMEM_SHARED((N_PAD, D), jnp.float32),
        pltpu.SemaphoreType.DMA,
    ],
)(_sc_body)


BM = 2000


def _mm_body(x_ref, w1_ref, b1_ref, w2_ref, b2_ref, o_ref):
    xb = x_ref[...]
    dims = (((1,), (1,)), ((), ()))
    o_ref[0] = lax.dot_general(
        xb, w1_ref[...], dims, preferred_element_type=jnp.float32) + b1_ref[...]
    o_ref[1] = lax.dot_general(
        xb, w2_ref[...], dims, preferred_element_type=jnp.float32) + b2_ref[...]


def _hidden(x, W1, b1, W2, b2):
    h = pl.pallas_call(
        _mm_body,
        grid=(N_NODES // BM,),
        in_specs=[
            pl.BlockSpec((BM, D), lambda i: (i, 0)),
            pl.BlockSpec((D, D), lambda i: (0, 0)),
            pl.BlockSpec((1, D), lambda i: (0, 0)),
            pl.BlockSpec((D, D), lambda i: (0, 0)),
            pl.BlockSpec((1, D), lambda i: (0, 0)),
        ],
        out_specs=pl.BlockSpec((2, BM, D), lambda i: (0, i, 0)),
        out_shape=jax.ShapeDtypeStruct((2, N_NODES, D), jnp.float32),
    )(x, W1, b1.reshape(1, D), W2, b2.reshape(1, D))
    return h.reshape(2 * N_NODES, D)


def _add_body(p_ref, o_ref):
    o_ref[...] = p_ref[0] + p_ref[1]


def _final_add(partials):
    return pl.pallas_call(
        _add_body,
        grid=(N_NODES // BM,),
        in_specs=[pl.BlockSpec((2, BM, D), lambda i: (0, i, 0))],
        out_specs=pl.BlockSpec((BM, D), lambda i: (i, 0)),
        out_shape=jax.ShapeDtypeStruct((N_NODES, D), jnp.float32),
    )(partials)


def kernel(x, edge_index1, edge_weight1, edge_index2, edge_weight2,
           W1, b1, W2, b2):
    pad = EC_PAD - 2 * E_EDGES
    zpad_i = jnp.zeros((pad,), jnp.int32)
    rows = jnp.concatenate(
        [edge_index1[0], edge_index2[0], zpad_i]).reshape(NW, NCHUNK, CHUNK)
    cols = jnp.concatenate(
        [edge_index1[1], edge_index2[1] + N_NODES,
         zpad_i]).reshape(NW, NCHUNK, CHUNK)
    w = jnp.concatenate(
        [edge_weight1, edge_weight2,
         jnp.zeros((pad,), jnp.float32)]).reshape(NW, NCHUNK, CHUNK)
    H = _hidden(x, W1, b1, W2, b2)
    partials = _sc_aggregate(H, rows, cols, w)
    return _final_add(partials)


# speedup vs baseline: 2.4453x; 1.1390x over previous
import functools

import jax
import jax.numpy as jnp
from jax import lax
from jax.experimental import pallas as pl
from jax.experimental.pallas import tpu as pltpu
from jax.experimental.pallas import tpu_sc as plsc

N_NODES = 10000
D = 128
E_EDGES = 320000
NC, NS = 2, 16
NW = NC * NS
CHUNK = 128
NCHUNK = 160
SEG = 32
NSEG = NCHUNK // SEG
E_PER = NCHUNK * CHUNK
EC_PAD = NW * E_PER
N_PAD = 10240
ROWS_PER_TILE = N_PAD // NS
LANES = 16


def _sc_body(h_hbm, row_hbm, col_hbm, w_hbm, out_hbm,
             row_v, col_v, w_v, gbuf, gbuf1, acc, sem, sem1):
    c = lax.axis_index("c")
    s = lax.axis_index("s")
    wid = c * NS + s

    zero = jnp.zeros((LANES,), jnp.float32)

    def zrow(i, carry):
        for j in range(D // LANES):
            gbuf[i, pl.ds(j * LANES, LANES)] = zero
        return carry

    lax.fori_loop(0, CHUNK, zrow, 0)
    for z in range(ROWS_PER_TILE // CHUNK):
        pltpu.sync_copy(
            gbuf, acc.at[pl.ds(s * ROWS_PER_TILE + z * CHUNK, CHUNK)])
    plsc.subcore_barrier()

    bufs = (gbuf, gbuf1)
    sems = (sem, sem1)

    def scale_and_scatter(gb, k):
        for b in range(CHUNK // LANES):
            wv16 = w_v[k, pl.ds(b * LANES, LANES)]
            for l in range(LANES):
                wv = jnp.full((LANES,), wv16[l], jnp.float32)
                r = b * LANES + l
                for j in range(D // LANES):
                    gb[r, pl.ds(j * LANES, LANES)] = (
                        gb[r, pl.ds(j * LANES, LANES)] * wv)
        pltpu.sync_copy(gb, acc.at[row_v.at[k]], add=True)

    def seg_body(g, carry):
        pltpu.sync_copy(row_hbm.at[wid, pl.ds(g * SEG, SEG)], row_v)
        pltpu.sync_copy(col_hbm.at[wid, pl.ds(g * SEG, SEG)], col_v)
        pltpu.sync_copy(w_hbm.at[wid, pl.ds(g * SEG, SEG)], w_v)

        pltpu.async_copy(h_hbm.at[col_v.at[0]], gbuf, sems[0])

        def pair_body(k2, inner):
            for b in range(2):
                kk = k2 * 2 + b
                pltpu.make_async_copy(
                    h_hbm.at[col_v.at[kk]], bufs[b], sems[b]).wait()
                if b == 0:
                    pltpu.async_copy(
                        h_hbm.at[col_v.at[kk + 1]], bufs[1], sems[1])
                else:
                    @pl.when(kk + 1 < SEG)
                    def _():
                        pltpu.async_copy(
                            h_hbm.at[col_v.at[kk + 1]], bufs[0], sems[0])
                scale_and_scatter(bufs[b], kk)
            return inner

        lax.fori_loop(0, SEG // 2, pair_body, 0)
        return carry

    lax.fori_loop(0, NSEG, seg_body, 0)

    plsc.subcore_barrier()
    pltpu.sync_copy(acc.at[pl.ds(s * ROWS_PER_TILE, ROWS_PER_TILE)],
                    out_hbm.at[c, pl.ds(s * ROWS_PER_TILE, ROWS_PER_TILE)])


_sc_aggregate = functools.partial(
    pl.kernel,
    out_type=jax.ShapeDtypeStruct((NC, N_PAD, D), jnp.float32),
    mesh=plsc.VectorSubcoreMesh(core_axis_name="c", subcore_axis_name="s"),
    scratch_types=[
        pltpu.VMEM((SEG, CHUNK), jnp.int32),
        pltpu.VMEM((SEG, CHUNK), jnp.int32),
        pltpu.VMEM((SEG, CHUNK), jnp.float32),
        pltpu.VMEM((CHUNK, D), jnp.float32),
        pltpu.VMEM((CHUNK, D), jnp.float32),
        pltpu.VMEM_SHARED((N_PAD, D), jnp.float32),
        pltpu.SemaphoreType.DMA,
        pltpu.SemaphoreType.DMA,
    ],
)(_sc_body)


BM = 2000


def _mm_body(x_ref, w1_ref, b1_ref, w2_ref, b2_ref, o_ref):
    xb = x_ref[...]
    dims = (((1,), (1,)), ((), ()))
    o_ref[0] = lax.dot_general(
        xb, w1_ref[...], dims, preferred_element_type=jnp.float32) + b1_ref[...]
    o_ref[1] = lax.dot_general(
        xb, w2_ref[...], dims, preferred_element_type=jnp.float32) + b2_ref[...]


def _hidden(x, W1, b1, W2, b2):
    h = pl.pallas_call(
        _mm_body,
        grid=(N_NODES // BM,),
        in_specs=[
            pl.BlockSpec((BM, D), lambda i: (i, 0)),
            pl.BlockSpec((D, D), lambda i: (0, 0)),
            pl.BlockSpec((1, D), lambda i: (0, 0)),
            pl.BlockSpec((D, D), lambda i: (0, 0)),
            pl.BlockSpec((1, D), lambda i: (0, 0)),
        ],
        out_specs=pl.BlockSpec((2, BM, D), lambda i: (0, i, 0)),
        out_shape=jax.ShapeDtypeStruct((2, N_NODES, D), jnp.float32),
    )(x, W1, b1.reshape(1, D), W2, b2.reshape(1, D))
    return h.reshape(2 * N_NODES, D)


def _add_body(p_ref, o_ref):
    o_ref[...] = p_ref[0] + p_ref[1]


def _final_add(partials):
    return pl.pallas_call(
        _add_body,
        grid=(N_NODES // BM,),
        in_specs=[pl.BlockSpec((2, BM, D), lambda i: (0, i, 0))],
        out_specs=pl.BlockSpec((BM, D), lambda i: (i, 0)),
        out_shape=jax.ShapeDtypeStruct((N_NODES, D), jnp.float32),
    )(partials)


def kernel(x, edge_index1, edge_weight1, edge_index2, edge_weight2,
           W1, b1, W2, b2):
    pad = EC_PAD - 2 * E_EDGES
    zpad_i = jnp.zeros((pad,), jnp.int32)
    rows = jnp.concatenate(
        [edge_index1[0], edge_index2[0], zpad_i]).reshape(NW, NCHUNK, CHUNK)
    cols = jnp.concatenate(
        [edge_index1[1], edge_index2[1] + N_NODES,
         zpad_i]).reshape(NW, NCHUNK, CHUNK)
    w = jnp.concatenate(
        [edge_weight1, edge_weight2,
         jnp.zeros((pad,), jnp.float32)]).reshape(NW, NCHUNK, CHUNK)
    H = _hidden(x, W1, b1, W2, b2)
    partials = _sc_aggregate(H, rows, cols, w)
    return _final_add(partials)


# speedup vs baseline: 6.7028x; 2.7411x over previous
import functools

import jax
import jax.numpy as jnp
from jax import lax
from jax.experimental import pallas as pl
from jax.experimental.pallas import tpu as pltpu
from jax.experimental.pallas import tpu_sc as plsc

N_NODES = 10000
D = 128
E_EDGES = 320000
NC, NS = 2, 16
NW = NC * NS
CHUNK = 128
NCHUNK = 160
SEG = 32
NSEG = NCHUNK // SEG
E_PER = NCHUNK * CHUNK
EC_PAD = NW * E_PER
N_PAD = 10240
ROWS_PER_TILE = N_PAD // NS
LANES = 16


def _sc_body(h_hbm, row_hbm, col_hbm, w_hbm, out_hbm,
             row_v, col_v, w_v, gbuf, gbuf1, acc, sem, sem1):
    c = lax.axis_index("c")
    s = lax.axis_index("s")
    wid = c * NS + s

    zero = jnp.zeros((LANES,), jnp.float32)

    def zrow(i, carry):
        for j in range(D // LANES):
            gbuf[i, pl.ds(j * LANES, LANES)] = zero
        return carry

    lax.fori_loop(0, CHUNK, zrow, 0)
    for z in range(ROWS_PER_TILE // CHUNK):
        pltpu.sync_copy(
            gbuf, acc.at[pl.ds(s * ROWS_PER_TILE + z * CHUNK, CHUNK)])
    plsc.subcore_barrier()

    bufs = (gbuf, gbuf1)
    sems = (sem, sem1)

    def scale_and_scatter(gb, k):
        for b in range(CHUNK // LANES):
            wv16 = w_v[k, pl.ds(b * LANES, LANES)]
            for l in range(LANES):
                wv = jnp.full((LANES,), wv16[l], jnp.float32)
                r = b * LANES + l
                for j in range(D // LANES):
                    gb[r, pl.ds(j * LANES, LANES)] = (
                        gb[r, pl.ds(j * LANES, LANES)] * wv)
        pltpu.sync_copy(gb, acc.at[row_v.at[k]], add=True)

    def seg_body(g, carry):
        pltpu.sync_copy(row_hbm.at[wid, pl.ds(g * SEG, SEG)], row_v)
        pltpu.sync_copy(col_hbm.at[wid, pl.ds(g * SEG, SEG)], col_v)
        pltpu.sync_copy(w_hbm.at[wid, pl.ds(g * SEG, SEG)], w_v)

        pltpu.async_copy(h_hbm.at[col_v.at[0]], gbuf, sems[0])

        def pair_body(k2, inner):
            for b in range(2):
                kk = k2 * 2 + b
                pltpu.make_async_copy(
                    h_hbm.at[col_v.at[kk]], bufs[b], sems[b]).wait()
                if b == 0:
                    pltpu.async_copy(
                        h_hbm.at[col_v.at[kk + 1]], bufs[1], sems[1])
                else:
                    @pl.when(kk + 1 < SEG)
                    def _():
                        pltpu.async_copy(
                            h_hbm.at[col_v.at[kk + 1]], bufs[0], sems[0])
                scale_and_scatter(bufs[b], kk)
            return inner

        lax.fori_loop(0, SEG // 2, pair_body, 0)
        return carry

    lax.fori_loop(0, NSEG, seg_body, 0)

    plsc.subcore_barrier()
    pltpu.sync_copy(acc.at[pl.ds(s * ROWS_PER_TILE, ROWS_PER_TILE)],
                    out_hbm.at[c, pl.ds(s * ROWS_PER_TILE, ROWS_PER_TILE)])


_sc_aggregate = functools.partial(
    pl.kernel,
    out_type=jax.ShapeDtypeStruct((NC, N_PAD, D), jnp.float32),
    mesh=plsc.VectorSubcoreMesh(core_axis_name="c", subcore_axis_name="s"),
    scratch_types=[
        pltpu.VMEM((SEG, CHUNK), jnp.int32),
        pltpu.VMEM((SEG, CHUNK), jnp.int32),
        pltpu.VMEM((SEG, CHUNK), jnp.float32),
        pltpu.VMEM((CHUNK, D), jnp.float32),
        pltpu.VMEM((CHUNK, D), jnp.float32),
        pltpu.VMEM_SHARED((N_PAD, D), jnp.float32),
        pltpu.SemaphoreType.DMA,
        pltpu.SemaphoreType.DMA,
    ],
)(_sc_body)


BM = 2000


def _mm_body(x_ref, w1_ref, b1_ref, w2_ref, b2_ref, o_ref):
    xb = x_ref[...]
    dims = (((1,), (1,)), ((), ()))
    o_ref[0] = lax.dot_general(
        xb, w1_ref[...], dims, preferred_element_type=jnp.float32) + b1_ref[...]
    o_ref[1] = lax.dot_general(
        xb, w2_ref[...], dims, preferred_element_type=jnp.float32) + b2_ref[...]


def _hidden(x, W1, b1, W2, b2):
    h = pl.pallas_call(
        _mm_body,
        grid=(N_NODES // BM,),
        in_specs=[
            pl.BlockSpec((BM, D), lambda i: (i, 0)),
            pl.BlockSpec((D, D), lambda i: (0, 0)),
            pl.BlockSpec((1, D), lambda i: (0, 0)),
            pl.BlockSpec((D, D), lambda i: (0, 0)),
            pl.BlockSpec((1, D), lambda i: (0, 0)),
        ],
        out_specs=pl.BlockSpec((2, BM, D), lambda i: (0, i, 0)),
        out_shape=jax.ShapeDtypeStruct((2, N_NODES, D), jnp.float32),
    )(x, W1, b1.reshape(1, D), W2, b2.reshape(1, D))
    return h.reshape(2 * N_NODES, D)


def _add_body(p_ref, o_ref):
    o_ref[...] = p_ref[0] + p_ref[1]


def _final_add(partials):
    return pl.pallas_call(
        _add_body,
        grid=(N_NODES // BM,),
        in_specs=[pl.BlockSpec((2, BM, D), lambda i: (0, i, 0))],
        out_specs=pl.BlockSpec((BM, D), lambda i: (i, 0)),
        out_shape=jax.ShapeDtypeStruct((N_NODES, D), jnp.float32),
    )(partials)


def kernel(x, edge_index1, edge_weight1, edge_index2, edge_weight2,
           W1, b1, W2, b2):
    pad = EC_PAD - 2 * E_EDGES
    spread = jnp.arange(pad, dtype=jnp.int32)
    rows = jnp.concatenate(
        [edge_index1[0], edge_index2[0],
         spread % N_PAD]).reshape(NW, NCHUNK, CHUNK)
    cols = jnp.concatenate(
        [edge_index1[1], edge_index2[1] + N_NODES,
         spread % (2 * N_NODES)]).reshape(NW, NCHUNK, CHUNK)
    w = jnp.concatenate(
        [edge_weight1, edge_weight2,
         jnp.zeros((pad,), jnp.float32)]).reshape(NW, NCHUNK, CHUNK)
    H = _hidden(x, W1, b1, W2, b2)
    partials = _sc_aggregate(H, rows, cols, w)
    return _final_add(partials)
